# Initial kernel scaffold; baseline (speedup 1.0000x reference)
import jax
import jax.numpy as jnp
from jax.experimental import pallas as pl


def kernel(atom_types, residue_types, extra_feats, W_atom, W_res):
    raise NotImplementedError("write your pallas kernel here")


# speedup vs baseline: 1.0679x; 1.0679x over previous
import functools

import jax
import jax.numpy as jnp
from jax import lax
from jax.experimental import pallas as pl
from jax.experimental.pallas import tpu as pltpu
from jax.experimental.pallas import tpu_sc as plsc

N = 100000
D_ATOM = 32
D_RES = 16
D_NUM = 16
D_OUT = D_ATOM + D_RES + D_NUM

NC, NS = 2, 16
NW = NC * NS
SUB = 80
NSUB = 5
C = SUB * NSUB
NCHUNKS = N // C
BIG = NCHUNKS - NW * (NCHUNKS // NW)


def _body(atom_idx_hbm, res_idx_hbm, feats_hbm, wa_hbm, wr_hbm, out_hbm,
          idx_a, idx_r, rows_a, rows_r, feats_v, sem):
    wid = lax.axis_index("s") * NC + lax.axis_index("c")
    base_chunks = NCHUNKS // NW
    start = wid * base_chunks + jnp.minimum(wid, BIG)
    count = jnp.where(wid < BIG, base_chunks + 1, base_chunks)

    def chunk_body(chunk, carry):
        base = chunk * C
        pltpu.sync_copy(atom_idx_hbm.at[chunk], idx_a)
        pltpu.sync_copy(res_idx_hbm.at[chunk], idx_r)
        pltpu.sync_copy(feats_hbm.at[pl.ds(base, C)], feats_v)
        for j in range(NSUB):
            pltpu.async_copy(wa_hbm.at[idx_a.at[j]],
                             rows_a.at[pl.ds(SUB * j, SUB)], sem).wait()
            pltpu.async_copy(wr_hbm.at[idx_r.at[j]],
                             rows_r.at[pl.ds(SUB * j, SUB)], sem).wait()
        pltpu.sync_copy(rows_a, out_hbm.at[pl.ds(base, C), pl.ds(0, D_ATOM)])
        pltpu.sync_copy(rows_r,
                        out_hbm.at[pl.ds(base, C), pl.ds(D_ATOM, D_RES)])
        pltpu.sync_copy(feats_v,
                        out_hbm.at[pl.ds(base, C),
                                   pl.ds(D_ATOM + D_RES, D_NUM)])
        return carry

    lax.fori_loop(start, start + count, chunk_body, 0)


@jax.jit
def _sc_embed(atom_idx3, res_idx3, extra_feats, W_atom, W_res):
    mesh = plsc.VectorSubcoreMesh(core_axis_name="c", subcore_axis_name="s",
                                  num_cores=NC, num_subcores=NS)
    f = functools.partial(
        pl.kernel,
        out_type=jax.ShapeDtypeStruct((N, D_OUT), jnp.float32),
        mesh=mesh,
        scratch_types=[
            pltpu.VMEM((NSUB, SUB), jnp.int32),
            pltpu.VMEM((NSUB, SUB), jnp.int32),
            pltpu.VMEM((C, D_ATOM), jnp.float32),
            pltpu.VMEM((C, D_RES), jnp.float32),
            pltpu.VMEM((C, D_NUM), jnp.float32),
            pltpu.SemaphoreType.DMA,
        ],
        compiler_params=pltpu.CompilerParams(use_tc_tiling_on_sc=False),
    )(_body)
    return f(atom_idx3, res_idx3, extra_feats, W_atom, W_res)


def kernel(atom_types, residue_types, extra_feats, W_atom, W_res):
    atom_idx3 = atom_types.reshape(NCHUNKS, NSUB, SUB)
    res_idx3 = residue_types.reshape(NCHUNKS, NSUB, SUB)
    return _sc_embed(atom_idx3, res_idx3, extra_feats, W_atom, W_res)


# speedup vs baseline: 1.2585x; 1.1784x over previous
import functools

import jax
import jax.numpy as jnp
from jax import lax
from jax.experimental import pallas as pl
from jax.experimental.pallas import tpu as pltpu
from jax.experimental.pallas import tpu_sc as plsc

N = 100000
D_ATOM = 32
D_RES = 16
D_NUM = 16
D_OUT = D_ATOM + D_RES + D_NUM

NC, NS = 2, 16
NW = NC * NS
SUB = 80
NSUB = 5
C = SUB * NSUB
NCHUNKS = N // C
BIG = NCHUNKS - NW * (NCHUNKS // NW)


def _body(atom_idx_hbm, res_idx_hbm, feats_hbm, wa_hbm, wr_hbm, out_hbm,
          idx_a, idx_r, rows_a, rows_r, feats_v, sem):
    wid = lax.axis_index("s") * NC + lax.axis_index("c")
    base_chunks = NCHUNKS // NW
    start = wid * base_chunks + jnp.minimum(wid, BIG)
    count = jnp.where(wid < BIG, base_chunks + 1, base_chunks)

    def chunk_body(chunk, carry):
        base = chunk * C
        pltpu.sync_copy(atom_idx_hbm.at[pl.ds(base, C)], idx_a)
        pltpu.sync_copy(res_idx_hbm.at[pl.ds(base, C)], idx_r)
        pltpu.sync_copy(feats_hbm.at[pl.ds(base, C)], feats_v)
        copies = []
        for j in range(NSUB):
            copies.append(
                pltpu.async_copy(wa_hbm.at[idx_a.at[pl.ds(SUB * j, SUB)]],
                                 rows_a.at[pl.ds(SUB * j, SUB)], sem))
            copies.append(
                pltpu.async_copy(wr_hbm.at[idx_r.at[pl.ds(SUB * j, SUB)]],
                                 rows_r.at[pl.ds(SUB * j, SUB)], sem))
        for c in copies:
            c.wait()
        pltpu.sync_copy(rows_a, out_hbm.at[pl.ds(base, C), pl.ds(0, D_ATOM)])
        pltpu.sync_copy(rows_r,
                        out_hbm.at[pl.ds(base, C), pl.ds(D_ATOM, D_RES)])
        pltpu.sync_copy(feats_v,
                        out_hbm.at[pl.ds(base, C),
                                   pl.ds(D_ATOM + D_RES, D_NUM)])
        return carry

    lax.fori_loop(start, start + count, chunk_body, 0)


@jax.jit
def _sc_embed(atom_types, residue_types, extra_feats, W_atom, W_res):
    mesh = plsc.VectorSubcoreMesh(core_axis_name="c", subcore_axis_name="s",
                                  num_cores=NC, num_subcores=NS)
    f = functools.partial(
        pl.kernel,
        out_type=jax.ShapeDtypeStruct((N, D_OUT), jnp.float32),
        mesh=mesh,
        scratch_types=[
            pltpu.VMEM((C,), jnp.int32),
            pltpu.VMEM((C,), jnp.int32),
            pltpu.VMEM((C, D_ATOM), jnp.float32),
            pltpu.VMEM((C, D_RES), jnp.float32),
            pltpu.VMEM((C, D_NUM), jnp.float32),
            pltpu.SemaphoreType.DMA,
        ],
        compiler_params=pltpu.CompilerParams(use_tc_tiling_on_sc=False),
    )(_body)
    return f(atom_types, residue_types, extra_feats, W_atom, W_res)


def kernel(atom_types, residue_types, extra_feats, W_atom, W_res):
    return _sc_embed(atom_types, residue_types, extra_feats, W_atom, W_res)
